# Initial kernel scaffold; baseline (speedup 1.0000x reference)
import functools

import jax
import jax.numpy as jnp
from jax import lax
from jax.experimental import pallas as pl
from jax.experimental.pallas import tpu as pltpu
from jax.experimental.pallas import tpu_sc as plsc

N = 10000
E = 320000
D = 128
R = 8

NC = 2
NS = 16
NW = NC * NS
EP = E // NW
C = 400
NCHUNK = EP // C
ROWS_PER_TILE = N // NS

_L = 16


_BN = 500


def _xrel_body(x_ref, w_ref, o_ref):
    xb = x_ref[...]
    for r in range(R):
        o_ref[:, r, :] = jnp.dot(xb, w_ref[r], preferred_element_type=jnp.float32)


def _compute_xrel(x, W):
    return pl.pallas_call(
        _xrel_body,
        grid=(N // _BN,),
        in_specs=[
            pl.BlockSpec((_BN, D), lambda i: (i, 0)),
            pl.BlockSpec((R, D, D), lambda i: (0, 0, 0)),
        ],
        out_specs=pl.BlockSpec((_BN, R, D), lambda i: (i, 0, 0)),
        out_shape=jax.ShapeDtypeStruct((N, R, D), jnp.float32),
    )(x, W)


def _sc_body(xrel_hbm, src_hbm, dst_hbm, et_hbm, norm_hbm, out_hbm,
             src_v, dst_v, gidx_v, norm_v, rows_v, h_acc, sem):
    cid = lax.axis_index("c")
    sid = lax.axis_index("s")
    wid = sid * NC + cid
    ebase = wid * EP

    zero = jnp.zeros((_L,), jnp.float32)

    def zbody(i, _):
        for f in range(D // _L):
            rows_v[i, pl.ds(f * _L, _L)] = zero
        return 0

    lax.fori_loop(0, C, zbody, 0)
    pltpu.sync_copy(rows_v.at[pl.ds(0, 400)],
                    h_acc.at[pl.ds(sid * ROWS_PER_TILE, 400)])
    pltpu.sync_copy(rows_v.at[pl.ds(0, 225)],
                    h_acc.at[pl.ds(sid * ROWS_PER_TILE + 400, 225)])
    plsc.subcore_barrier()

    def chunk_body(k, _):
        eoff = ebase + k * C
        pltpu.sync_copy(src_hbm.at[pl.ds(eoff, C)], src_v)
        pltpu.sync_copy(et_hbm.at[pl.ds(eoff, C)], gidx_v)
        pltpu.sync_copy(dst_hbm.at[pl.ds(eoff, C)], dst_v)
        pltpu.sync_copy(norm_hbm.at[pl.ds(eoff, C)], norm_v)

        def gbody(g, _):
            s = src_v[pl.ds(g * _L, _L)]
            e = gidx_v[pl.ds(g * _L, _L)]
            gidx_v[pl.ds(g * _L, _L)] = s * R + e
            return 0

        lax.fori_loop(0, C // _L, gbody, 0)

        pltpu.async_copy(xrel_hbm.at[gidx_v], rows_v, sem).wait()

        def sbody(g, _):
            base = g * _L
            for e in range(_L):
                s = norm_v[base + e]
                for f in range(D // _L):
                    sl = pl.ds(f * _L, _L)
                    rows_v[base + e, sl] = rows_v[base + e, sl] * s
            return 0

        lax.fori_loop(0, C // _L, sbody, 0)

        pltpu.sync_copy(rows_v, h_acc.at[dst_v], add=True)
        return 0

    lax.fori_loop(0, NCHUNK, chunk_body, 0)

    plsc.subcore_barrier()
    pltpu.sync_copy(h_acc.at[pl.ds(sid * ROWS_PER_TILE, ROWS_PER_TILE)],
                    out_hbm.at[cid, pl.ds(sid * ROWS_PER_TILE, ROWS_PER_TILE)])


_sc_kernel = functools.partial(
    pl.kernel,
    out_type=jax.ShapeDtypeStruct((NC, N, D), jnp.float32),
    mesh=plsc.VectorSubcoreMesh(core_axis_name="c", subcore_axis_name="s"),
    scratch_types=[
        pltpu.VMEM((C,), jnp.int32),
        pltpu.VMEM((C,), jnp.int32),
        pltpu.VMEM((C,), jnp.int32),
        pltpu.VMEM((C,), jnp.float32),
        pltpu.VMEM((C, D), jnp.float32),
        pltpu.VMEM_SHARED((N, D), jnp.float32),
        pltpu.SemaphoreType.DMA,
    ],
)(_sc_body)


_BS = 1000


def _sum_body(p_ref, o_ref):
    o_ref[...] = p_ref[0] + p_ref[1]


def _sum_partials(partials):
    return pl.pallas_call(
        _sum_body,
        grid=(N // _BS,),
        in_specs=[pl.BlockSpec((NC, _BS, D), lambda i: (0, i, 0))],
        out_specs=pl.BlockSpec((_BS, D), lambda i: (i, 0)),
        out_shape=jax.ShapeDtypeStruct((N, D), jnp.float32),
    )(partials)


@jax.jit
def kernel(x, W, edge_index, edge_type, edge_norm):
    x_rel = _compute_xrel(x, W).reshape(N * R, D)
    src = edge_index[0]
    dst = edge_index[1]
    partials = _sc_kernel(x_rel, src, dst, edge_type, edge_norm)
    return _sum_partials(partials)


# speedup vs baseline: 12.6618x; 12.6618x over previous
import functools

import jax
import jax.numpy as jnp
from jax import lax
from jax.experimental import pallas as pl
from jax.experimental.pallas import tpu as pltpu
from jax.experimental.pallas import tpu_sc as plsc

N = 10000
E = 320000
D = 128
R = 8

NC = 2
NS = 16
NW = NC * NS
EP = E // NW
C = 80
NCHUNK = EP // C
ROWS_PER_TILE = 624
TAIL_BASE = NS * ROWS_PER_TILE
TAIL_ROWS = N - TAIL_BASE

_L = 16


_BN = 1000


def _xrel_body(x_ref, w_ref, o_ref):
    xb = x_ref[...]
    for r in range(R):
        o_ref[:, r, :] = jnp.dot(xb, w_ref[r], preferred_element_type=jnp.float32)


def _compute_xrel(x, W):
    return pl.pallas_call(
        _xrel_body,
        grid=(N // _BN,),
        in_specs=[
            pl.BlockSpec((_BN, D), lambda i: (i, 0)),
            pl.BlockSpec((R, D, D), lambda i: (0, 0, 0)),
        ],
        out_specs=pl.BlockSpec((_BN, R, D), lambda i: (i, 0, 0)),
        out_shape=jax.ShapeDtypeStruct((N, R, D), jnp.float32),
    )(x, W)


def _sc_body(xrel_hbm, src_hbm, dst_hbm, et_hbm, norm_hbm, out_hbm,
             src_v, dst_v, gidx_v, norm_v, rows_v, h_acc, sem):
    cid = lax.axis_index("c")
    sid = lax.axis_index("s")
    wid = sid * NC + cid
    ebase = wid * EP

    zero = jnp.zeros((_L,), jnp.float32)

    def zbody(i, _):
        for f in range(D // _L):
            rows_v[i, pl.ds(f * _L, _L)] = zero
        return 0

    lax.fori_loop(0, C, zbody, 0)

    def zcopy(i, _):
        pltpu.sync_copy(rows_v,
                        h_acc.at[pl.ds(sid * ROWS_PER_TILE + i * C, C)])
        return 0

    lax.fori_loop(0, 8, zcopy, 0)
    plsc.subcore_barrier()

    def chunk_body(k, _):
        eoff = ebase + k * C
        pltpu.sync_copy(src_hbm.at[pl.ds(eoff, C)], src_v)
        pltpu.sync_copy(et_hbm.at[pl.ds(eoff, C)], gidx_v)
        pltpu.sync_copy(dst_hbm.at[pl.ds(eoff, C)], dst_v)
        pltpu.sync_copy(norm_hbm.at[pl.ds(eoff, C)], norm_v)

        def gbody(g, _):
            s = src_v[pl.ds(g * _L, _L)]
            e = gidx_v[pl.ds(g * _L, _L)]
            gidx_v[pl.ds(g * _L, _L)] = s * R + e
            return 0

        lax.fori_loop(0, C // _L, gbody, 0)

        pltpu.async_copy(xrel_hbm.at[gidx_v], rows_v, sem).wait()

        def sbody(g, _):
            base = g * _L
            nv = norm_v[pl.ds(base, _L)]
            for e in range(_L):
                s = nv[e]
                for f in range(D // _L):
                    sl = pl.ds(f * _L, _L)
                    rows_v[base + e, sl] = rows_v[base + e, sl] * s
            return 0

        lax.fori_loop(0, C // _L, sbody, 0)

        pltpu.sync_copy(rows_v, h_acc.at[dst_v], add=True)
        return 0

    lax.fori_loop(0, NCHUNK, chunk_body, 0)

    plsc.subcore_barrier()
    pltpu.sync_copy(h_acc.at[pl.ds(sid * ROWS_PER_TILE, ROWS_PER_TILE)],
                    out_hbm.at[cid, pl.ds(sid * ROWS_PER_TILE, ROWS_PER_TILE)])

    @pl.when(sid == NS - 1)
    def _write_tail():
        pltpu.sync_copy(h_acc.at[pl.ds(TAIL_BASE, TAIL_ROWS)],
                        out_hbm.at[cid, pl.ds(TAIL_BASE, TAIL_ROWS)])


_sc_kernel = functools.partial(
    pl.kernel,
    out_type=jax.ShapeDtypeStruct((NC, N, D), jnp.float32),
    mesh=plsc.VectorSubcoreMesh(core_axis_name="c", subcore_axis_name="s"),
    scratch_types=[
        pltpu.VMEM((C,), jnp.int32),
        pltpu.VMEM((C,), jnp.int32),
        pltpu.VMEM((C,), jnp.int32),
        pltpu.VMEM((C,), jnp.float32),
        pltpu.VMEM((C, D), jnp.float32),
        pltpu.VMEM_SHARED((N, D), jnp.float32),
        pltpu.SemaphoreType.DMA,
    ],
)(_sc_body)


_BS = 1000


def _sum_body(p_ref, o_ref):
    o_ref[...] = p_ref[0] + p_ref[1]


def _sum_partials(partials):
    return pl.pallas_call(
        _sum_body,
        grid=(N // _BS,),
        in_specs=[pl.BlockSpec((NC, _BS, D), lambda i: (0, i, 0))],
        out_specs=pl.BlockSpec((_BS, D), lambda i: (i, 0)),
        out_shape=jax.ShapeDtypeStruct((N, D), jnp.float32),
    )(partials)


@jax.jit
def kernel(x, W, edge_index, edge_type, edge_norm):
    x_rel = _compute_xrel(x, W).reshape(N * R, D)
    src = edge_index[0]
    dst = edge_index[1]
    partials = _sc_kernel(x_rel, src, dst, edge_type, edge_norm)
    return _sum_partials(partials)


# speedup vs baseline: 18.8813x; 1.4912x over previous
import functools

import jax
import jax.numpy as jnp
from jax import lax
from jax.experimental import pallas as pl
from jax.experimental.pallas import tpu as pltpu
from jax.experimental.pallas import tpu_sc as plsc

N = 10000
E = 320000
D = 128
R = 8

NC = 2
NS = 16
NW = NC * NS
EP = E // NW
C = 80
NCHUNK = EP // C
ROWS_PER_TILE = 624
TAIL_BASE = NS * ROWS_PER_TILE
TAIL_ROWS = N - TAIL_BASE

_L = 16


_BN = 1000


def _xrel_body(x_ref, w_ref, o_ref):
    xb = x_ref[...]
    for r in range(R):
        o_ref[:, r, :] = jnp.dot(xb, w_ref[r], preferred_element_type=jnp.float32)


def _compute_xrel(x, W):
    return pl.pallas_call(
        _xrel_body,
        grid=(N // _BN,),
        in_specs=[
            pl.BlockSpec((_BN, D), lambda i: (i, 0)),
            pl.BlockSpec((R, D, D), lambda i: (0, 0, 0)),
        ],
        out_specs=pl.BlockSpec((_BN, R, D), lambda i: (i, 0, 0)),
        out_shape=jax.ShapeDtypeStruct((N, R, D), jnp.float32),
    )(x, W)


def _sc_body(xrel_hbm, src_hbm, dst_hbm, et_hbm, norm_hbm, out_hbm,
             src_t,
             rows0, rows1, rows2, rows3,
             gid0, gid1, gid2, gid3,
             dstv0, dstv1, dstv2, dstv3,
             nrm0, nrm1, nrm2, nrm3,
             h_acc,
             gs0, gs1, gs2, gs3, ss0, ss1, ss2, ss3):
    rows = (rows0, rows1, rows2, rows3)
    gid = (gid0, gid1, gid2, gid3)
    dstv = (dstv0, dstv1, dstv2, dstv3)
    nrm = (nrm0, nrm1, nrm2, nrm3)
    gsem = (gs0, gs1, gs2, gs3)
    ssem = (ss0, ss1, ss2, ss3)

    cid = lax.axis_index("c")
    sid = lax.axis_index("s")
    wid = sid * NC + cid
    ebase = wid * EP

    zero = jnp.zeros((_L,), jnp.float32)

    def zbody(i, _):
        for f in range(D // _L):
            rows0[i, pl.ds(f * _L, _L)] = zero
        return 0

    lax.fori_loop(0, C, zbody, 0)

    def zcopy(i, _):
        pltpu.sync_copy(rows0,
                        h_acc.at[pl.ds(sid * ROWS_PER_TILE + i * C, C)])
        return 0

    lax.fori_loop(0, 8, zcopy, 0)
    plsc.subcore_barrier()

    def prefetch(c, b, drain):
        if drain:
            pltpu.make_async_copy(rows[b], h_acc.at[dstv[b]], ssem[b]).wait()
        eoff = ebase + c * C
        pltpu.sync_copy(src_hbm.at[pl.ds(eoff, C)], src_t)
        pltpu.sync_copy(et_hbm.at[pl.ds(eoff, C)], gid[b])
        pltpu.sync_copy(dst_hbm.at[pl.ds(eoff, C)], dstv[b])
        pltpu.sync_copy(norm_hbm.at[pl.ds(eoff, C)], nrm[b])

        def gbody(g, _):
            s = src_t[pl.ds(g * _L, _L)]
            e = gid[b][pl.ds(g * _L, _L)]
            gid[b][pl.ds(g * _L, _L)] = s * R + e
            return 0

        lax.fori_loop(0, C // _L, gbody, 0)
        pltpu.async_copy(xrel_hbm.at[gid[b]], rows[b], gsem[b])

    def process(b):
        pltpu.make_async_copy(xrel_hbm.at[gid[b]], rows[b], gsem[b]).wait()

        def sbody(g, _):
            base = g * _L
            nv = nrm[b][pl.ds(base, _L)]
            for e in range(_L):
                s = nv[e]
                for f in range(D // _L):
                    sl = pl.ds(f * _L, _L)
                    rows[b][base + e, sl] = rows[b][base + e, sl] * s
            return 0

        lax.fori_loop(0, C // _L, sbody, 0)
        pltpu.async_copy(rows[b], h_acc.at[dstv[b]], ssem[b], add=True)

    prefetch(0, 0, False)
    prefetch(1, 1, False)
    process(0)
    prefetch(2, 2, False)
    process(1)
    prefetch(3, 3, False)

    def quad(q, _):
        c0 = 2 + q * 4
        for j in range(4):
            b = (2 + j) % 4
            process(b)
            prefetch(c0 + j + 2, (b + 2) % 4, True)
        return 0

    lax.fori_loop(0, 30, quad, 0)

    process(2)
    prefetch(NCHUNK - 1, 0, True)
    process(3)
    process(0)

    for b in range(4):
        pltpu.make_async_copy(rows[b], h_acc.at[dstv[b]], ssem[b]).wait()

    plsc.subcore_barrier()
    pltpu.sync_copy(h_acc.at[pl.ds(sid * ROWS_PER_TILE, ROWS_PER_TILE)],
                    out_hbm.at[cid, pl.ds(sid * ROWS_PER_TILE, ROWS_PER_TILE)])

    @pl.when(sid == NS - 1)
    def _write_tail():
        pltpu.sync_copy(h_acc.at[pl.ds(TAIL_BASE, TAIL_ROWS)],
                        out_hbm.at[cid, pl.ds(TAIL_BASE, TAIL_ROWS)])


_sc_kernel = functools.partial(
    pl.kernel,
    out_type=jax.ShapeDtypeStruct((NC, N, D), jnp.float32),
    mesh=plsc.VectorSubcoreMesh(core_axis_name="c", subcore_axis_name="s"),
    scratch_types=(
        [pltpu.VMEM((C,), jnp.int32)]
        + [pltpu.VMEM((C, D), jnp.float32)] * 4
        + [pltpu.VMEM((C,), jnp.int32)] * 4
        + [pltpu.VMEM((C,), jnp.int32)] * 4
        + [pltpu.VMEM((C,), jnp.float32)] * 4
        + [pltpu.VMEM_SHARED((N, D), jnp.float32)]
        + [pltpu.SemaphoreType.DMA] * 8
    ),
)(_sc_body)


_BS = 1000


def _sum_body(p_ref, o_ref):
    o_ref[...] = p_ref[0] + p_ref[1]


def _sum_partials(partials):
    return pl.pallas_call(
        _sum_body,
        grid=(N // _BS,),
        in_specs=[pl.BlockSpec((NC, _BS, D), lambda i: (0, i, 0))],
        out_specs=pl.BlockSpec((_BS, D), lambda i: (i, 0)),
        out_shape=jax.ShapeDtypeStruct((N, D), jnp.float32),
    )(partials)


@jax.jit
def kernel(x, W, edge_index, edge_type, edge_norm):
    x_rel = _compute_xrel(x, W).reshape(N * R, D)
    src = edge_index[0]
    dst = edge_index[1]
    partials = _sc_kernel(x_rel, src, dst, edge_type, edge_norm)
    return _sum_partials(partials)
